# Initial kernel scaffold; baseline (speedup 1.0000x reference)
import jax
import jax.numpy as jnp
from jax.experimental import pallas as pl


def kernel(target_features, feature_dist_graph, rij_dist_pairs, b_scope, start_end_env, l_scope, scope_update, scope_update_lig, W_emb, b_emb, W_dist, b_dist, W_gat, b_gat, a_vec, W_upd):
    raise NotImplementedError("write your pallas kernel here")


# speedup vs baseline: 5.0155x; 5.0155x over previous
import functools
import jax
import jax.numpy as jnp
from jax import lax
from jax.experimental import pallas as pl
from jax.experimental.pallas import tpu as pltpu
from jax.experimental.pallas import tpu_sc as plsc

N = 50000
E = 800000
H = 32
FEAT = 128
ED = 16
P = N + 1
B = 512
L = 32

NC = 2
NS = 16
NW = NC * NS

P2 = 50176
Q = P2 // 4
Q2 = 12672
QPT = Q2 // NS
PAYW = 128
ECH = 200
EPW = E // NW
NFULL = EPW // ECH

_mesh = plsc.VectorSubcoreMesh(core_axis_name="c", subcore_axis_name="s")


GW = 128
GCH = 1000


def _gather_rows(table, idx, n_total):
    per_w = n_total // NW
    nchunks = per_w // GCH
    rem = per_w - nchunks * GCH
    ch = GCH if nchunks > 0 else rem

    @functools.partial(
        pl.kernel, mesh=_mesh,
        out_type=jax.ShapeDtypeStruct((n_total, GW), jnp.float32),
        scratch_types=[pltpu.VMEM((ch,), jnp.int32),
                       pltpu.VMEM((ch, GW), jnp.float32),
                       pltpu.SemaphoreType.DMA],
    )
    def k(table_hbm, idx_hbm, out_hbm, idx_a, rows_a, sem):
        wid = lax.axis_index("s") * NC + lax.axis_index("c")
        base = wid * per_w

        def body(j, carry):
            off = base + j * ch
            pltpu.sync_copy(idx_hbm.at[pl.ds(off, ch)], idx_a)
            pltpu.async_copy(table_hbm.at[idx_a], rows_a, sem).wait()
            pltpu.sync_copy(rows_a, out_hbm.at[pl.ds(off, ch)])
            return carry

        if nchunks > 1:
            lax.fori_loop(0, nchunks, body, 0)
        else:
            body(0, 0)

    return k(table, idx)


def _scatter_quarter(pay, idxq, zeros_tile):
    scratch = [
        pltpu.VMEM((ECH,), jnp.int32),
        pltpu.VMEM((ECH, PAYW), jnp.float32),
        pltpu.VMEM_SHARED((Q2, PAYW), jnp.float32),
    ]

    @functools.partial(
        pl.kernel, mesh=_mesh,
        out_type=jax.ShapeDtypeStruct((NC, Q2, PAYW), jnp.float32),
        scratch_types=scratch,
    )
    def k(pay_hbm, idx_hbm, zeros_hbm, out_hbm, idx_a, pay_a, acc):
        cid = lax.axis_index("c")
        sid = lax.axis_index("s")
        wid = sid * NC + cid
        pltpu.sync_copy(zeros_hbm, acc.at[pl.ds(sid * QPT, QPT)])
        plsc.subcore_barrier()
        base = wid * EPW

        def body(j, carry):
            off = base + j * ECH
            pltpu.sync_copy(idx_hbm.at[pl.ds(off, ECH)], idx_a)
            pltpu.sync_copy(pay_hbm.at[pl.ds(off, ECH)], pay_a)
            pltpu.sync_copy(pay_a, acc.at[idx_a], add=True)
            return carry

        lax.fori_loop(0, NFULL, body, 0)
        plsc.subcore_barrier()
        r0 = sid * QPT
        pltpu.sync_copy(acc.at[pl.ds(r0, QPT)],
                        out_hbm.at[cid, pl.ds(r0, QPT)])

    return k(pay, idxq, zeros_tile)


def _segment_softmax_sum(pay, idx4, zeros_tile):
    outs = [_scatter_quarter(pay, idx4[q], zeros_tile) for q in range(4)]
    return jnp.stack(outs, axis=0)


EBLK = 2000
NBLK = 2000
PBLK = P2 // 16


def _embed_nodes(tf, W_emb, b_emb):
    def body(x_ref, w_ref, b_ref, o_ref):
        o_ref[...] = jnp.dot(x_ref[...], w_ref[...],
                             preferred_element_type=jnp.float32) + b_ref[...]
    return pl.pallas_call(
        body,
        grid=(N // NBLK,),
        in_specs=[pl.BlockSpec((NBLK, FEAT), lambda i: (i, 0)),
                  pl.BlockSpec((FEAT, H), lambda i: (0, 0)),
                  pl.BlockSpec((1, H), lambda i: (0, 0))],
        out_specs=pl.BlockSpec((NBLK, H), lambda i: (i, 0)),
        out_shape=jax.ShapeDtypeStruct((N, H), jnp.float32),
    )(tf, W_emb, b_emb.reshape(1, H))


def _embed_msg(fdg, rij, Wd_a, wd_r, b_dist):
    def body(f_ref, r_ref, wa_ref, wr_ref, b_ref, o_ref):
        o_ref[...] = (jnp.dot(f_ref[...], wa_ref[...],
                              preferred_element_type=jnp.float32)
                      + r_ref[...] * wr_ref[...] + b_ref[...])
    return pl.pallas_call(
        body,
        grid=(E // EBLK,),
        in_specs=[pl.BlockSpec((EBLK, ED), lambda i: (i, 0)),
                  pl.BlockSpec((EBLK, 1), lambda i: (i, 0)),
                  pl.BlockSpec((ED, H), lambda i: (0, 0)),
                  pl.BlockSpec((1, H), lambda i: (0, 0)),
                  pl.BlockSpec((1, H), lambda i: (0, 0))],
        out_specs=pl.BlockSpec((EBLK, H), lambda i: (i, 0)),
        out_shape=jax.ShapeDtypeStruct((E, H), jnp.float32),
    )(fdg, rij.reshape(E, 1), Wd_a, wd_r, b_dist.reshape(1, H))


def _attn_body(nbr, msg, w1_ref, w2_ref, bg_ref, av_ref, pay_ref):
    z = (jnp.dot(nbr, w1_ref[...], preferred_element_type=jnp.float32)
         + jnp.dot(msg, w2_ref[...], preferred_element_type=jnp.float32)
         + bg_ref[...])
    e = jnp.sum(z * av_ref[...], axis=1, keepdims=True)
    e = jnp.where(e >= 0, e, 0.2 * e)
    s = jnp.exp(e)
    pay_ref[...] = jnp.concatenate(
        [s * z, s, jnp.zeros((z.shape[0], PAYW - H - 1), jnp.float32)], axis=1)


def _attn1(nbr, msg, W1, W2, b_gat, a_vec):
    def body(n_ref, m_ref, w1, w2, bg, av, pay_ref):
        _attn_body(n_ref[:, :H], m_ref[...], w1, w2, bg, av, pay_ref)
    return pl.pallas_call(
        body,
        grid=(E // EBLK,),
        in_specs=[pl.BlockSpec((EBLK, GW), lambda i: (i, 0)),
                  pl.BlockSpec((EBLK, H), lambda i: (i, 0)),
                  pl.BlockSpec((H, H), lambda i: (0, 0)),
                  pl.BlockSpec((H, H), lambda i: (0, 0)),
                  pl.BlockSpec((1, H), lambda i: (0, 0)),
                  pl.BlockSpec((1, H), lambda i: (0, 0))],
        out_specs=pl.BlockSpec((EBLK, PAYW), lambda i: (i, 0)),
        out_shape=jax.ShapeDtypeStruct((E, PAYW), jnp.float32),
    )(nbr, msg, W1, W2, b_gat.reshape(1, H), a_vec.reshape(1, H))


def _attn2(u1, u2, nbr, msg0, Wu1, Wu2, W1, W2, b_gat, a_vec):
    def body(u1_ref, u2_ref, n_ref, m_ref, wu1, wu2, w1, w2, bg, av,
             pay_ref):
        pre = (jnp.dot(u1_ref[:, :H], wu1[...],
                       preferred_element_type=jnp.float32)
               + jnp.dot(u2_ref[:, :H], wu2[...],
                         preferred_element_type=jnp.float32)
               + m_ref[...])
        msg1 = jnp.where(pre > 0, pre, jnp.exp(pre) - 1.0)
        _attn_body(n_ref[:, :H], msg1, w1, w2, bg, av, pay_ref)
    return pl.pallas_call(
        body,
        grid=(E // EBLK,),
        in_specs=[pl.BlockSpec((EBLK, GW), lambda i: (i, 0)),
                  pl.BlockSpec((EBLK, GW), lambda i: (i, 0)),
                  pl.BlockSpec((EBLK, GW), lambda i: (i, 0)),
                  pl.BlockSpec((EBLK, H), lambda i: (i, 0)),
                  pl.BlockSpec((H, H), lambda i: (0, 0)),
                  pl.BlockSpec((H, H), lambda i: (0, 0)),
                  pl.BlockSpec((H, H), lambda i: (0, 0)),
                  pl.BlockSpec((H, H), lambda i: (0, 0)),
                  pl.BlockSpec((1, H), lambda i: (0, 0)),
                  pl.BlockSpec((1, H), lambda i: (0, 0))],
        out_specs=pl.BlockSpec((EBLK, PAYW), lambda i: (i, 0)),
        out_shape=jax.ShapeDtypeStruct((E, PAYW), jnp.float32),
    )(u1, u2, nbr, msg0, Wu1, Wu2, W1, W2, b_gat.reshape(1, H),
      a_vec.reshape(1, H))


QB = 784


def _combine_norm(nd):
    def body(nd_ref, o_ref):
        x = nd_ref[0]
        num = x[0, :, :H] + x[1, :, :H]
        den = x[0, :, H:H + 1] + x[1, :, H:H + 1]
        v = num / (den + 1e-9)
        v = jnp.where(v > 0, v, jnp.exp(v) - 1.0)
        o_ref[...] = jnp.concatenate(
            [v, jnp.zeros((v.shape[0], GW - H), jnp.float32)], axis=1)
    return pl.pallas_call(
        body,
        grid=(4, Q // QB),
        in_specs=[pl.BlockSpec((1, NC, QB, PAYW), lambda q, b: (q, 0, b, 0))],
        out_specs=pl.BlockSpec((QB, GW), lambda q, b: (q * (Q // QB) + b, 0)),
        out_shape=jax.ShapeDtypeStruct((P2, GW), jnp.float32),
    )(nd)


def _readout_sum(cmp_enc):
    def body(x_ref, o_ref):
        o_ref[...] = jnp.sum(x_ref[:, :, :H], axis=1)
    return pl.pallas_call(
        body,
        grid=(1,),
        in_specs=[pl.BlockSpec((B, L, GW), lambda i: (0, 0, 0))],
        out_specs=pl.BlockSpec((B, H), lambda i: (0, 0)),
        out_shape=jax.ShapeDtypeStruct((B, H), jnp.float32),
    )(cmp_enc)


def kernel(target_features, feature_dist_graph, rij_dist_pairs, b_scope,
           start_end_env, l_scope, scope_update, scope_update_lig,
           W_emb, b_emb, W_dist, b_dist, W_gat, b_gat, a_vec, W_upd):
    W1 = W_gat[:H]
    W2 = W_gat[H:]
    Wu1 = W_upd[:H]
    Wu2 = W_upd[H:]
    Wd_a = W_dist[:ED]
    wd_r = W_dist[ED:ED + 1]
    zeros_tile = jnp.zeros((QPT, PAYW), jnp.float32)
    idx4 = jnp.stack([
        jnp.where((b_scope >= q * Q) & (b_scope < (q + 1) * Q),
                  b_scope - q * Q, Q)
        for q in range(4)], axis=0)

    h_core = _embed_nodes(target_features, W_emb, b_emb)
    h = jnp.zeros((P2, GW), jnp.float32).at[1:N + 1, :H].set(h_core)
    msg0 = _embed_msg(feature_dist_graph, rij_dist_pairs, Wd_a, wd_r, b_dist)

    nbr = _gather_rows(h, start_end_env, E)
    pay = _attn1(nbr, msg0, W1, W2, b_gat, a_vec)
    nd = _segment_softmax_sum(pay, idx4, zeros_tile)
    h1 = _combine_norm(nd)

    u1 = _gather_rows(h1, scope_update, E)
    u2 = _gather_rows(h1, scope_update_lig, E)
    nbr2 = _gather_rows(h1, start_end_env, E)
    pay2 = _attn2(u1, u2, nbr2, msg0, Wu1, Wu2, W1, W2, b_gat, a_vec)
    nd2 = _segment_softmax_sum(pay2, idx4, zeros_tile)
    h2 = _combine_norm(nd2)

    cmp_flat = _gather_rows(h2, l_scope.reshape(B * L), B * L)
    mol_vecs = _readout_sum(cmp_flat.reshape(B, L, GW))
    return mol_vecs
